# baseline (speedup 1.0000x reference)
import functools

import jax
import jax.numpy as jnp
from jax import lax
from jax.experimental import pallas as pl
from jax.experimental.pallas import tpu as pltpu
from jax.experimental.pallas import tpu_sc as plsc

_CHUNK = 128
_NTILES = 32
_NPAD = 10240
_NCH = 80


def _edge_scatter_kernel(n, d, e):
    rows_per_tile = n // 16

    mesh = plsc.VectorSubcoreMesh(core_axis_name="c", subcore_axis_name="s")

    @functools.partial(
        pl.kernel,
        out_type=jax.ShapeDtypeStruct((2 * n, d), jnp.float32),
        mesh=mesh,
        scratch_types=[
            pltpu.VMEM((8, 2, _CHUNK), jnp.int32),
            pltpu.VMEM((_CHUNK, d), jnp.float32),
            pltpu.VMEM_SHARED((n, d), jnp.float32),
            pltpu.SemaphoreType.DMA,
        ],
    )
    def body(idx_hbm, hs_hbm, zeros_hbm, out_hbm, idxb, rows, acc, sem):
        cid = lax.axis_index("c")
        sid = lax.axis_index("s")
        wid = sid * 2 + cid
        row0 = sid * rows_per_tile

        @pl.when(cid == 0)
        def _():
            pltpu.sync_copy(hs_hbm.at[pl.ds(row0, rows_per_tile)],
                            acc.at[pl.ds(row0, rows_per_tile)])

        @pl.when(cid != 0)
        def _():
            pltpu.sync_copy(zeros_hbm.at[pl.ds(row0, rows_per_tile)],
                            acc.at[pl.ds(row0, rows_per_tile)])

        plsc.subcore_barrier()

        def grp(g, carry):
            pltpu.sync_copy(idx_hbm.at[wid, g], idxb)
            for j in range(8):
                pltpu.async_copy(hs_hbm.at[idxb.at[j, 0]], rows, sem).wait()
                pltpu.sync_copy(rows, acc.at[idxb.at[j, 1]], add=True)
            return carry

        lax.fori_loop(0, _NCH // 8, grp, 0)

        plsc.subcore_barrier()
        pltpu.sync_copy(acc.at[pl.ds(row0, rows_per_tile)],
                        out_hbm.at[pl.ds(cid * n + row0, rows_per_tile)])

    return body


def _deg_kernel():
    stripe = _NPAD // 16

    mesh = plsc.VectorSubcoreMesh(core_axis_name="c", subcore_axis_name="s")

    @functools.partial(
        pl.kernel,
        out_type=jax.ShapeDtypeStruct((2 * _NPAD,), jnp.float32),
        mesh=mesh,
        scratch_types=[
            pltpu.VMEM((_NCH, _CHUNK), jnp.int32),
            pltpu.VMEM((_CHUNK,), jnp.float32),
            pltpu.VMEM_SHARED((_NPAD,), jnp.float32),
            pltpu.SemaphoreType.DMA,
        ],
    )
    def body(dst_hbm, zeros_hbm, out_hbm, didx, ones, acc, sem):
        cid = lax.axis_index("c")
        sid = lax.axis_index("s")
        wid = sid * 2 + cid
        row0 = sid * stripe

        for i in range(_CHUNK // 16):
            ones[pl.ds(i * 16, 16)] = jnp.full((16,), 1.0, jnp.float32)

        pltpu.sync_copy(zeros_hbm.at[pl.ds(row0, stripe)],
                        acc.at[pl.ds(row0, stripe)])
        pltpu.sync_copy(dst_hbm.at[wid], didx)
        plsc.subcore_barrier()

        def step(k, carry):
            pltpu.async_copy(ones, acc.at[didx.at[k]], sem, add=True)
            return carry

        lax.fori_loop(0, _NCH, step, 0)
        pltpu.make_async_copy(dst_hbm.at[wid], didx, sem).wait()

        plsc.subcore_barrier()
        pltpu.sync_copy(acc.at[pl.ds(row0, stripe)],
                        out_hbm.at[pl.ds(cid * _NPAD + row0, stripe)])

    return body


def _tc_first(degb, x, w1):
    n, din = x.shape
    dh = w1.shape[1]
    blk = 2048
    grid = n // blk

    def body(deg_ref, x_ref, w_ref, hs_ref, dinv_ref):
        deg = deg_ref[...]
        d = deg[:, 0:1] + deg[:, 1:2] + 1.0
        dinv = lax.rsqrt(d)
        h = jnp.dot(x_ref[...], w_ref[...], preferred_element_type=jnp.float32)
        hs_ref[...] = h * dinv
        dinv_ref[...] = dinv

    return pl.pallas_call(
        body,
        grid=(grid,),
        in_specs=[
            pl.BlockSpec((blk, 2), lambda i: (i, 0)),
            pl.BlockSpec((blk, din), lambda i: (i, 0)),
            pl.BlockSpec((din, dh), lambda i: (0, 0)),
        ],
        out_specs=[
            pl.BlockSpec((blk, dh), lambda i: (i, 0)),
            pl.BlockSpec((blk, 1), lambda i: (i, 0)),
        ],
        out_shape=[
            jax.ShapeDtypeStruct((n, dh), jnp.float32),
            jax.ShapeDtypeStruct((n, 1), jnp.float32),
        ],
    )(degb, x, w1)


def _tc_mid(p0, p1, dinv, b1, w2):
    n, dh = p0.shape
    blk = 2048
    grid = n // blk

    def body(p0_ref, p1_ref, dinv_ref, b_ref, w_ref, hs_ref):
        dinv = dinv_ref[...]
        t = jnp.maximum(dinv * (p0_ref[...] + p1_ref[...]) + b_ref[...], 0.0)
        h = jnp.dot(t, w_ref[...], preferred_element_type=jnp.float32)
        hs_ref[...] = h * dinv

    return pl.pallas_call(
        body,
        grid=(grid,),
        in_specs=[
            pl.BlockSpec((blk, dh), lambda i: (i, 0)),
            pl.BlockSpec((blk, dh), lambda i: (i, 0)),
            pl.BlockSpec((blk, 1), lambda i: (i, 0)),
            pl.BlockSpec((1, dh), lambda i: (0, 0)),
            pl.BlockSpec((dh, dh), lambda i: (0, 0)),
        ],
        out_specs=pl.BlockSpec((blk, dh), lambda i: (i, 0)),
        out_shape=jax.ShapeDtypeStruct((n, dh), jnp.float32),
    )(p0, p1, dinv, b1, w2)


def _tc_final(p0, p1, dinv, b2, batch2, wc, bc, n_graphs):
    n, dh = p0.shape
    ncls = wc.shape[1]
    blk = 2048
    grid = n // blk

    def body(p0_ref, p1_ref, dinv_ref, b_ref, batch_ref, wc_ref, bc_ref,
             out_ref, sums, cnt):
        pid = pl.program_id(0)

        @pl.when(pid == 0)
        def _():
            sums[...] = jnp.zeros_like(sums)
            cnt[...] = jnp.zeros_like(cnt)

        dinv = dinv_ref[...]
        t = jnp.maximum(dinv * (p0_ref[...] + p1_ref[...]) + b_ref[...], 0.0)
        seg = batch_ref[...]
        onehot = (seg == lax.broadcasted_iota(jnp.int32, (1, n_graphs), 1))
        onehot = onehot.astype(jnp.float32)
        sums[...] += lax.dot_general(
            onehot, t, (((0,), (0,)), ((), ())),
            preferred_element_type=jnp.float32)
        c = jnp.sum(onehot, axis=0)[:, None]
        cnt[...] += jnp.broadcast_to(c, cnt.shape)

        @pl.when(pid == grid - 1)
        def _():
            pooled = sums[...] / jnp.maximum(cnt[...], 1.0)
            out_ref[...] = (
                jnp.dot(pooled, wc_ref[...],
                        preferred_element_type=jnp.float32) + bc_ref[...])

    return pl.pallas_call(
        body,
        grid=(grid,),
        in_specs=[
            pl.BlockSpec((blk, dh), lambda i: (i, 0)),
            pl.BlockSpec((blk, dh), lambda i: (i, 0)),
            pl.BlockSpec((blk, 1), lambda i: (i, 0)),
            pl.BlockSpec((1, dh), lambda i: (0, 0)),
            pl.BlockSpec((blk, 1), lambda i: (i, 0)),
            pl.BlockSpec((dh, ncls), lambda i: (0, 0)),
            pl.BlockSpec((1, ncls), lambda i: (0, 0)),
        ],
        out_specs=pl.BlockSpec((n_graphs, ncls), lambda i: (0, 0)),
        out_shape=jax.ShapeDtypeStruct((n_graphs, ncls), jnp.float32),
        scratch_shapes=[
            pltpu.VMEM((n_graphs, dh), jnp.float32),
            pltpu.VMEM((n_graphs, dh), jnp.float32),
        ],
    )(p0, p1, dinv, b2, batch2, wc, bc)


def kernel(x, edge_index, batch, W1, b1, W2, b2, Wc, bc):
    n, din = x.shape
    e = edge_index.shape[1]
    dh = W1.shape[1]
    n_graphs = 64
    np_ = _NPAD

    src = edge_index[0]
    dst = edge_index[1]
    ep = _NTILES * _NCH * _CHUNK
    dstp = jnp.pad(dst, (0, ep - e), constant_values=np_ - 1)
    dst3 = dstp.reshape(_NTILES, _NCH, _CHUNK)

    xp = jnp.pad(x, ((0, np_ - n), (0, 0)))
    batchp = jnp.pad(batch, (0, np_ - n), constant_values=n_graphs)
    zeros2d = jnp.zeros((np_, dh), jnp.float32)
    zeros1 = jnp.zeros((np_,), jnp.float32)

    degp = _deg_kernel()(dst3, zeros1)
    degb = degp.reshape(2, np_).T

    hs1, dinv = _tc_first(degb, xp, W1)

    edge_fn = _edge_scatter_kernel(np_, dh, e)

    srcp = jnp.pad(src, (0, ep - e), constant_values=np_ - 1)
    idx3 = jnp.stack([srcp, dstp]).reshape(2, _NTILES, _NCH // 8, 8, _CHUNK)
    idx3 = idx3.transpose(1, 2, 3, 0, 4)

    s1 = edge_fn(idx3, hs1, zeros2d)
    hs2 = _tc_mid(s1[:np_], s1[np_:], dinv, b1.reshape(1, dh), W2)

    s2 = edge_fn(idx3, hs2, zeros2d)
    logits = _tc_final(s2[:np_], s2[np_:], dinv, b2.reshape(1, dh),
                       batchp.reshape(np_, 1), Wc, bc.reshape(1, -1), n_graphs)
    return logits


# speedup vs baseline: 2.3085x; 2.3085x over previous
import functools

import jax
import jax.numpy as jnp
from jax import lax
from jax.experimental import pallas as pl
from jax.experimental.pallas import tpu as pltpu
from jax.experimental.pallas import tpu_sc as plsc

_CHUNK = 128
_NTILES = 32
_NPAD = 10240
_NCH = 80


def _edge_scatter_kernel(n, d, e):
    nch_total = e // _CHUNK
    nch_base = nch_total // _NTILES
    nch_rem = nch_total % _NTILES
    rows_per_tile = n // 16

    mesh = plsc.VectorSubcoreMesh(core_axis_name="c", subcore_axis_name="s")

    @functools.partial(
        pl.kernel,
        out_type=jax.ShapeDtypeStruct((2 * n, d), jnp.float32),
        mesh=mesh,
        scratch_types=[
            pltpu.VMEM((2, _CHUNK), jnp.int32),
            pltpu.VMEM((_CHUNK, d), jnp.float32),
            pltpu.VMEM_SHARED((n, d), jnp.float32),
            pltpu.SemaphoreType.DMA,
        ],
    )
    def body(idx_hbm, hs_hbm, zeros_hbm, out_hbm, idxb, rows, acc, sem):
        cid = lax.axis_index("c")
        sid = lax.axis_index("s")
        wid = sid * 2 + cid
        row0 = sid * rows_per_tile

        @pl.when(cid == 0)
        def _():
            pltpu.sync_copy(hs_hbm.at[pl.ds(row0, rows_per_tile)],
                            acc.at[pl.ds(row0, rows_per_tile)])

        @pl.when(cid != 0)
        def _():
            pltpu.sync_copy(zeros_hbm.at[pl.ds(row0, rows_per_tile)],
                            acc.at[pl.ds(row0, rows_per_tile)])

        plsc.subcore_barrier()

        nch = nch_base + jnp.where(wid < nch_rem, 1, 0)

        def step(k, carry):
            c = wid + _NTILES * k
            pltpu.sync_copy(idx_hbm.at[c], idxb)
            pltpu.async_copy(hs_hbm.at[idxb.at[0]], rows, sem).wait()
            pltpu.sync_copy(rows, acc.at[idxb.at[1]], add=True)
            return carry

        lax.fori_loop(0, nch, step, 0)

        plsc.subcore_barrier()
        pltpu.sync_copy(acc.at[pl.ds(row0, rows_per_tile)],
                        out_hbm.at[pl.ds(cid * n + row0, rows_per_tile)])

    return body


def _deg_kernel():
    stripe = _NPAD // 16

    mesh = plsc.VectorSubcoreMesh(core_axis_name="c", subcore_axis_name="s")

    @functools.partial(
        pl.kernel,
        out_type=jax.ShapeDtypeStruct((2 * _NPAD,), jnp.float32),
        mesh=mesh,
        scratch_types=[
            pltpu.VMEM((_NCH, _CHUNK), jnp.int32),
            pltpu.VMEM((_CHUNK,), jnp.float32),
            pltpu.VMEM_SHARED((_NPAD,), jnp.float32),
            pltpu.SemaphoreType.DMA,
        ],
    )
    def body(dst_hbm, zeros_hbm, out_hbm, didx, ones, acc, sem):
        cid = lax.axis_index("c")
        sid = lax.axis_index("s")
        wid = sid * 2 + cid
        row0 = sid * stripe

        for i in range(_CHUNK // 16):
            ones[pl.ds(i * 16, 16)] = jnp.full((16,), 1.0, jnp.float32)

        pltpu.sync_copy(zeros_hbm.at[pl.ds(row0, stripe)],
                        acc.at[pl.ds(row0, stripe)])
        pltpu.sync_copy(dst_hbm.at[wid], didx)
        plsc.subcore_barrier()

        def step(k, carry):
            pltpu.async_copy(ones, acc.at[didx.at[k]], sem, add=True)
            return carry

        lax.fori_loop(0, _NCH, step, 0)
        pltpu.make_async_copy(dst_hbm.at[wid], didx, sem).wait()

        plsc.subcore_barrier()
        pltpu.sync_copy(acc.at[pl.ds(row0, stripe)],
                        out_hbm.at[pl.ds(cid * _NPAD + row0, stripe)])

    return body


def _tc_first(degb, x, w1):
    n, din = x.shape
    dh = w1.shape[1]
    blk = 2048
    grid = n // blk

    def body(deg_ref, x_ref, w_ref, hs_ref, dinv_ref):
        deg = deg_ref[...]
        d = deg[:, 0:1] + deg[:, 1:2] + 1.0
        dinv = lax.rsqrt(d)
        h = jnp.dot(x_ref[...], w_ref[...], preferred_element_type=jnp.float32)
        hs_ref[...] = h * dinv
        dinv_ref[...] = dinv

    return pl.pallas_call(
        body,
        grid=(grid,),
        in_specs=[
            pl.BlockSpec((blk, 2), lambda i: (i, 0)),
            pl.BlockSpec((blk, din), lambda i: (i, 0)),
            pl.BlockSpec((din, dh), lambda i: (0, 0)),
        ],
        out_specs=[
            pl.BlockSpec((blk, dh), lambda i: (i, 0)),
            pl.BlockSpec((blk, 1), lambda i: (i, 0)),
        ],
        out_shape=[
            jax.ShapeDtypeStruct((n, dh), jnp.float32),
            jax.ShapeDtypeStruct((n, 1), jnp.float32),
        ],
    )(degb, x, w1)


def _tc_mid(p0, p1, dinv, b1, w2):
    n, dh = p0.shape
    blk = 2048
    grid = n // blk

    def body(p0_ref, p1_ref, dinv_ref, b_ref, w_ref, hs_ref):
        dinv = dinv_ref[...]
        t = jnp.maximum(dinv * (p0_ref[...] + p1_ref[...]) + b_ref[...], 0.0)
        h = jnp.dot(t, w_ref[...], preferred_element_type=jnp.float32)
        hs_ref[...] = h * dinv

    return pl.pallas_call(
        body,
        grid=(grid,),
        in_specs=[
            pl.BlockSpec((blk, dh), lambda i: (i, 0)),
            pl.BlockSpec((blk, dh), lambda i: (i, 0)),
            pl.BlockSpec((blk, 1), lambda i: (i, 0)),
            pl.BlockSpec((1, dh), lambda i: (0, 0)),
            pl.BlockSpec((dh, dh), lambda i: (0, 0)),
        ],
        out_specs=pl.BlockSpec((blk, dh), lambda i: (i, 0)),
        out_shape=jax.ShapeDtypeStruct((n, dh), jnp.float32),
    )(p0, p1, dinv, b1, w2)


def _tc_final(p0, p1, dinv, b2, batch2, wc, bc, n_graphs):
    n, dh = p0.shape
    ncls = wc.shape[1]
    blk = 2048
    grid = n // blk

    def body(p0_ref, p1_ref, dinv_ref, b_ref, batch_ref, wc_ref, bc_ref,
             out_ref, sums, cnt):
        pid = pl.program_id(0)

        @pl.when(pid == 0)
        def _():
            sums[...] = jnp.zeros_like(sums)
            cnt[...] = jnp.zeros_like(cnt)

        dinv = dinv_ref[...]
        t = jnp.maximum(dinv * (p0_ref[...] + p1_ref[...]) + b_ref[...], 0.0)
        seg = batch_ref[...]
        onehot = (seg == lax.broadcasted_iota(jnp.int32, (1, n_graphs), 1))
        onehot = onehot.astype(jnp.float32)
        sums[...] += lax.dot_general(
            onehot, t, (((0,), (0,)), ((), ())),
            preferred_element_type=jnp.float32)
        c = jnp.sum(onehot, axis=0)[:, None]
        cnt[...] += jnp.broadcast_to(c, cnt.shape)

        @pl.when(pid == grid - 1)
        def _():
            pooled = sums[...] / jnp.maximum(cnt[...], 1.0)
            out_ref[...] = (
                jnp.dot(pooled, wc_ref[...],
                        preferred_element_type=jnp.float32) + bc_ref[...])

    return pl.pallas_call(
        body,
        grid=(grid,),
        in_specs=[
            pl.BlockSpec((blk, dh), lambda i: (i, 0)),
            pl.BlockSpec((blk, dh), lambda i: (i, 0)),
            pl.BlockSpec((blk, 1), lambda i: (i, 0)),
            pl.BlockSpec((1, dh), lambda i: (0, 0)),
            pl.BlockSpec((blk, 1), lambda i: (i, 0)),
            pl.BlockSpec((dh, ncls), lambda i: (0, 0)),
            pl.BlockSpec((1, ncls), lambda i: (0, 0)),
        ],
        out_specs=pl.BlockSpec((n_graphs, ncls), lambda i: (0, 0)),
        out_shape=jax.ShapeDtypeStruct((n_graphs, ncls), jnp.float32),
        scratch_shapes=[
            pltpu.VMEM((n_graphs, dh), jnp.float32),
            pltpu.VMEM((n_graphs, dh), jnp.float32),
        ],
    )(p0, p1, dinv, b2, batch2, wc, bc)


def kernel(x, edge_index, batch, W1, b1, W2, b2, Wc, bc):
    n, din = x.shape
    e = edge_index.shape[1]
    dh = W1.shape[1]
    n_graphs = 64
    np_ = _NPAD

    src = edge_index[0]
    dst = edge_index[1]
    ep = _NTILES * _NCH * _CHUNK
    dstp = jnp.pad(dst, (0, ep - e), constant_values=np_ - 1)
    dst3 = dstp.reshape(_NTILES, _NCH, _CHUNK)

    xp = jnp.pad(x, ((0, np_ - n), (0, 0)))
    batchp = jnp.pad(batch, (0, np_ - n), constant_values=n_graphs)
    zeros2d = jnp.zeros((np_, dh), jnp.float32)
    zeros1 = jnp.zeros((np_,), jnp.float32)

    degp = _deg_kernel()(dst3, zeros1)
    degb = degp.reshape(2, np_).T

    hs1, dinv = _tc_first(degb, xp, W1)

    edge_fn = _edge_scatter_kernel(np_, dh, e)

    idx3 = jnp.stack([src, dst]).reshape(2, e // _CHUNK, _CHUNK)
    idx3 = idx3.transpose(1, 0, 2)

    s1 = edge_fn(idx3, hs1, zeros2d)
    hs2 = _tc_mid(s1[:np_], s1[np_:], dinv, b1.reshape(1, dh), W2)

    s2 = edge_fn(idx3, hs2, zeros2d)
    logits = _tc_final(s2[:np_], s2[np_:], dinv, b2.reshape(1, dh),
                       batchp.reshape(np_, 1), Wc, bc.reshape(1, -1), n_graphs)
    return logits
